# baseline (speedup 1.0000x reference)
import functools

import jax
import jax.numpy as jnp
from jax import lax
from jax.experimental import pallas as pl
from jax.experimental.pallas import tpu as pltpu
from jax.experimental.pallas import tpu_sc as plsc

_NC = 2
_NS = 16
_NW = _NC * _NS

_D_CHAR = 16
_L_CHARS = 16
_D_WORD = 128
_TN = 400


def _sc_gather(num_rows, row_dim, chunk, tc_tiling):
    per_w = num_rows // _NW
    n_chunks = per_w // chunk
    mesh = plsc.VectorSubcoreMesh(core_axis_name="c", subcore_axis_name="s")

    @functools.partial(
        pl.kernel,
        out_type=jax.ShapeDtypeStruct((num_rows, row_dim), jnp.float32),
        mesh=mesh,
        compiler_params=pltpu.CompilerParams(use_tc_tiling_on_sc=tc_tiling),
        scratch_types=[
            pltpu.VMEM((chunk,), jnp.int32),
            pltpu.VMEM((chunk,), jnp.int32),
            pltpu.VMEM((chunk, row_dim), jnp.float32),
            pltpu.VMEM((chunk, row_dim), jnp.float32),
            pltpu.SemaphoreType.DMA,
            pltpu.SemaphoreType.DMA,
            pltpu.SemaphoreType.DMA,
            pltpu.SemaphoreType.DMA,
        ],
    )
    def gather(idx_hbm, table_hbm, out_hbm, idx0, idx1, rows0, rows1,
               gsem0, gsem1, osem0, osem1):
        wid = lax.axis_index("s") * _NC + lax.axis_index("c")
        base = wid * per_w
        idx_b, rows_b = [idx0, idx1], [rows0, rows1]
        gsem, osem = [gsem0, gsem1], [osem0, osem1]
        h_g = [None, None]
        h_o = [None, None]
        pltpu.sync_copy(idx_hbm.at[pl.ds(base, chunk)], idx_b[0])
        h_g[0] = pltpu.async_copy(table_hbm.at[idx_b[0]], rows_b[0], gsem[0])
        for c in range(n_chunks):
            cur, nxt = c % 2, (c + 1) % 2
            h_g[cur].wait()
            if c + 1 < n_chunks:
                pltpu.sync_copy(
                    idx_hbm.at[pl.ds(base + (c + 1) * chunk, chunk)],
                    idx_b[nxt])
                if c >= 1:
                    h_o[nxt].wait()
                h_g[nxt] = pltpu.async_copy(
                    table_hbm.at[idx_b[nxt]], rows_b[nxt], gsem[nxt])
            h_o[cur] = pltpu.async_copy(
                rows_b[cur], out_hbm.at[pl.ds(base + c * chunk, chunk)],
                osem[cur])
        h_o[(n_chunks - 1) % 2].wait()
        if n_chunks > 1:
            h_o[n_chunks % 2].wait()

    return gather


def _band(W, k):
    O = W.shape[0]
    T = jnp.transpose(W, (2, 1, 0))
    Tz = jnp.concatenate([T, jnp.zeros((1, _D_CHAR, O), W.dtype)], axis=0)
    t = jnp.arange(_L_CHARS)[:, None]
    p = jnp.arange(_L_CHARS)[None, :]
    dk = p - t + k // 2
    idx = jnp.where((dk >= 0) & (dk < k), dk, k)
    return Tz[idx]


def _conv_body(ce_ref, wv_ref, wb_ref, out_ref):
    x2 = ce_ref[...].astype(jnp.bfloat16)
    x = x2.reshape(_TN, 2 * _D_WORD)
    acc = jnp.dot(x, wb_ref[...], preferred_element_type=jnp.float32)
    m = acc[:, :256]
    for t in range(1, _L_CHARS):
        m = jnp.maximum(m, acc[:, 256 * t:256 * (t + 1)])
    ch = jnp.maximum(m[:, :_D_WORD], m[:, _D_WORD:])
    res = wv_ref[...] + jnp.maximum(ch, jnp.float32(0))
    for b in range(_TN // 50):
        out_ref[b] = res[b * 50:(b + 1) * 50, :]


def kernel(X, X_char, word_table, char_table, W3, W5):
    B, S = X.shape
    N = B * S
    n_blk = N // _TN
    flat_words = X.reshape(N).astype(jnp.int32)

    chars8 = X_char.astype(jnp.int8).reshape(N * 8, 2)
    pids = lax.bitcast_convert_type(chars8, jnp.int16).astype(jnp.int32)

    padded = jnp.pad(char_table, ((0, 128), (0, 0)))
    o_part = jnp.repeat(char_table, 64, axis=0)
    parts = []
    for i in range(4):
        parts.append(jnp.tile(padded[i::4], (128, 1)))
        parts.append(o_part)
    pair_table = jnp.concatenate(parts, axis=1)
    pair_table = pair_table.reshape(4 * 8192, 2 * _D_CHAR)

    word_vecs = _sc_gather(N, _D_WORD, 400, True)(flat_words, word_table)
    pids = pids + jnp.max(word_vecs[0, :1]).astype(jnp.int32) * 0
    char_emb = _sc_gather(N * 8, 2 * _D_CHAR, 1600, False)(pids, pair_table)
    ce = char_emb.reshape(N * 2, _D_WORD)

    Wb = jnp.concatenate([_band(W3, 3), _band(W5, 5)], axis=-1)
    Wb = jnp.transpose(Wb, (1, 2, 0, 3)).reshape(256, _L_CHARS * 256)
    Wb = Wb.astype(jnp.bfloat16)

    out = pl.pallas_call(
        _conv_body,
        grid=(n_blk,),
        in_specs=[
            pl.BlockSpec((2 * _TN, _D_WORD), lambda i: (i, 0)),
            pl.BlockSpec((_TN, _D_WORD), lambda i: (i, 0)),
            pl.BlockSpec((2 * _D_WORD, _L_CHARS * 256), lambda i: (0, 0)),
        ],
        out_specs=pl.BlockSpec((_TN // 50, S, _D_WORD), lambda i: (i, 0, 0)),
        out_shape=jax.ShapeDtypeStruct((B, S, _D_WORD), jnp.float32),
    )(ce, word_vecs, Wb)

    return out


# speedup vs baseline: 1.5525x; 1.5525x over previous
import functools

import jax
import jax.numpy as jnp
from jax import lax
from jax.experimental import pallas as pl
from jax.experimental.pallas import tpu as pltpu
from jax.experimental.pallas import tpu_sc as plsc

_NC = 2
_NS = 16
_NW = _NC * _NS

_D_CHAR = 16
_L_CHARS = 16
_D_WORD = 128
_TN = 400


def _sc_gather(num_rows, row_dim, chunk, tc_tiling):
    per_w = num_rows // _NW
    n_chunks = per_w // chunk
    mesh = plsc.VectorSubcoreMesh(core_axis_name="c", subcore_axis_name="s")

    @functools.partial(
        pl.kernel,
        out_type=jax.ShapeDtypeStruct((num_rows, row_dim), jnp.float32),
        mesh=mesh,
        compiler_params=pltpu.CompilerParams(use_tc_tiling_on_sc=tc_tiling),
        scratch_types=[
            pltpu.VMEM((chunk,), jnp.int32),
            pltpu.VMEM((chunk,), jnp.int32),
            pltpu.VMEM((chunk, row_dim), jnp.float32),
            pltpu.VMEM((chunk, row_dim), jnp.float32),
            pltpu.SemaphoreType.DMA,
            pltpu.SemaphoreType.DMA,
            pltpu.SemaphoreType.DMA,
            pltpu.SemaphoreType.DMA,
        ],
    )
    def gather(idx_hbm, table_hbm, out_hbm, idx0, idx1, rows0, rows1,
               gsem0, gsem1, osem0, osem1):
        wid = lax.axis_index("s") * _NC + lax.axis_index("c")
        base = wid * per_w
        idx_b, rows_b = [idx0, idx1], [rows0, rows1]
        gsem, osem = [gsem0, gsem1], [osem0, osem1]
        h_g = [None, None]
        h_o = [None, None]
        pltpu.sync_copy(idx_hbm.at[pl.ds(base, chunk)], idx_b[0])
        h_g[0] = pltpu.async_copy(table_hbm.at[idx_b[0]], rows_b[0], gsem[0])
        for c in range(n_chunks):
            cur, nxt = c % 2, (c + 1) % 2
            h_g[cur].wait()
            if c + 1 < n_chunks:
                pltpu.sync_copy(
                    idx_hbm.at[pl.ds(base + (c + 1) * chunk, chunk)],
                    idx_b[nxt])
                if c >= 1:
                    h_o[nxt].wait()
                h_g[nxt] = pltpu.async_copy(
                    table_hbm.at[idx_b[nxt]], rows_b[nxt], gsem[nxt])
            h_o[cur] = pltpu.async_copy(
                rows_b[cur], out_hbm.at[pl.ds(base + c * chunk, chunk)],
                osem[cur])
        h_o[(n_chunks - 1) % 2].wait()
        if n_chunks > 1:
            h_o[n_chunks % 2].wait()

    return gather


def _band(W, k):
    O = W.shape[0]
    T = jnp.transpose(W, (2, 1, 0))
    Tz = jnp.concatenate([T, jnp.zeros((1, _D_CHAR, O), W.dtype)], axis=0)
    t = jnp.arange(_L_CHARS)[:, None]
    p = jnp.arange(_L_CHARS)[None, :]
    dk = p - t + k // 2
    idx = jnp.where((dk >= 0) & (dk < k), dk, k)
    return Tz[idx]


def _conv_body(ce_ref, wv_ref, wb_ref, out_ref):
    x2 = ce_ref[...].astype(jnp.bfloat16)
    x = x2.reshape(_TN, 2 * _D_WORD)
    acc = jnp.dot(x, wb_ref[...], preferred_element_type=jnp.float32)
    m = acc[:, :256]
    for t in range(1, _L_CHARS):
        m = jnp.maximum(m, acc[:, 256 * t:256 * (t + 1)])
    ch = jnp.maximum(m[:, :_D_WORD], m[:, _D_WORD:])
    res = wv_ref[...] + jnp.maximum(ch, jnp.float32(0))
    for b in range(_TN // 50):
        out_ref[b] = res[b * 50:(b + 1) * 50, :]


def kernel(X, X_char, word_table, char_table, W3, W5):
    B, S = X.shape
    N = B * S
    n_blk = N // _TN
    flat_words = X.reshape(N).astype(jnp.int32)

    xf = X_char.astype(jnp.float32).reshape(N // 8, 128)
    lane = jnp.arange(128)
    pair_of = jnp.arange(64)
    M = jnp.where(
        (lane[:, None] // 2) == pair_of[None, :],
        jnp.where(lane % 2 == 0, 1.0, 256.0)[:, None], 0.0)
    pids = jnp.dot(xf, M, preferred_element_type=jnp.float32)
    pids = pids.astype(jnp.int32).reshape(N * 8)

    padded = jnp.pad(char_table, ((0, 128), (0, 0)))
    o_part = jnp.repeat(char_table, 64, axis=0)
    parts = []
    for i in range(4):
        parts.append(jnp.tile(padded[i::4], (128, 1)))
        parts.append(o_part)
    pair_table = jnp.concatenate(parts, axis=1)
    pair_table = pair_table.reshape(4 * 8192, 2 * _D_CHAR)

    word_vecs = _sc_gather(N, _D_WORD, 400, True)(flat_words, word_table)
    char_emb = _sc_gather(N * 8, 2 * _D_CHAR, 1600, False)(pids, pair_table)
    ce = char_emb.reshape(N * 2, _D_WORD)

    Wb = jnp.concatenate([_band(W3, 3), _band(W5, 5)], axis=-1)
    Wb = jnp.transpose(Wb, (1, 2, 0, 3)).reshape(256, _L_CHARS * 256)
    Wb = Wb.astype(jnp.bfloat16)

    out = pl.pallas_call(
        _conv_body,
        grid=(n_blk,),
        in_specs=[
            pl.BlockSpec((2 * _TN, _D_WORD), lambda i: (i, 0)),
            pl.BlockSpec((_TN, _D_WORD), lambda i: (i, 0)),
            pl.BlockSpec((2 * _D_WORD, _L_CHARS * 256), lambda i: (0, 0)),
        ],
        out_specs=pl.BlockSpec((_TN // 50, S, _D_WORD), lambda i: (i, 0, 0)),
        out_shape=jax.ShapeDtypeStruct((B, S, _D_WORD), jnp.float32),
    )(ce, word_vecs, Wb)

    return out


# speedup vs baseline: 1.5616x; 1.0058x over previous
import functools

import jax
import jax.numpy as jnp
from jax import lax
from jax.experimental import pallas as pl
from jax.experimental.pallas import tpu as pltpu
from jax.experimental.pallas import tpu_sc as plsc

_NC = 2
_NS = 16
_NW = _NC * _NS

_D_CHAR = 16
_L_CHARS = 16
_D_WORD = 128
_TN = 400


def _sc_gather(num_rows, row_dim, chunk, tc_tiling):
    per_w = num_rows // _NW
    n_chunks = per_w // chunk
    mesh = plsc.VectorSubcoreMesh(core_axis_name="c", subcore_axis_name="s")

    @functools.partial(
        pl.kernel,
        out_type=jax.ShapeDtypeStruct((num_rows, row_dim), jnp.float32),
        mesh=mesh,
        compiler_params=pltpu.CompilerParams(use_tc_tiling_on_sc=tc_tiling),
        scratch_types=[
            pltpu.VMEM((chunk,), jnp.int32),
            pltpu.VMEM((chunk,), jnp.int32),
            pltpu.VMEM((chunk, row_dim), jnp.float32),
            pltpu.VMEM((chunk, row_dim), jnp.float32),
            pltpu.SemaphoreType.DMA,
            pltpu.SemaphoreType.DMA,
            pltpu.SemaphoreType.DMA,
            pltpu.SemaphoreType.DMA,
        ],
    )
    def gather(idx_hbm, table_hbm, out_hbm, idx0, idx1, rows0, rows1,
               gsem0, gsem1, osem0, osem1):
        wid = lax.axis_index("s") * _NC + lax.axis_index("c")
        base = wid * per_w
        idx_b, rows_b = [idx0, idx1], [rows0, rows1]
        gsem, osem = [gsem0, gsem1], [osem0, osem1]
        h_g = [None, None]
        h_o = [None, None]
        pltpu.sync_copy(idx_hbm.at[pl.ds(base, chunk)], idx_b[0])
        h_g[0] = pltpu.async_copy(table_hbm.at[idx_b[0]], rows_b[0], gsem[0])
        for c in range(n_chunks):
            cur, nxt = c % 2, (c + 1) % 2
            h_g[cur].wait()
            if c + 1 < n_chunks:
                pltpu.sync_copy(
                    idx_hbm.at[pl.ds(base + (c + 1) * chunk, chunk)],
                    idx_b[nxt])
                if c >= 1:
                    h_o[nxt].wait()
                h_g[nxt] = pltpu.async_copy(
                    table_hbm.at[idx_b[nxt]], rows_b[nxt], gsem[nxt])
            h_o[cur] = pltpu.async_copy(
                rows_b[cur], out_hbm.at[pl.ds(base + c * chunk, chunk)],
                osem[cur])
        h_o[(n_chunks - 1) % 2].wait()
        if n_chunks > 1:
            h_o[n_chunks % 2].wait()

    return gather


def _band(W, k):
    O = W.shape[0]
    T = jnp.transpose(W, (2, 1, 0))
    Tz = jnp.concatenate([T, jnp.zeros((1, _D_CHAR, O), W.dtype)], axis=0)
    t = jnp.arange(_L_CHARS)[:, None]
    p = jnp.arange(_L_CHARS)[None, :]
    dk = p - t + k // 2
    idx = jnp.where((dk >= 0) & (dk < k), dk, k)
    return Tz[idx]


_GROUPS = ((0, 96), (32, 160), (96, 224), (160, 256))


def _conv_body(ce_ref, wv_ref, w0_ref, w1_ref, w2_ref, w3_ref, out_ref):
    x = ce_ref[...].astype(jnp.bfloat16).reshape(_TN, 2 * _D_WORD)
    ms = []
    for wr, (lo, hi) in zip((w0_ref, w1_ref, w2_ref, w3_ref), _GROUPS):
        acc = jnp.dot(x[:, lo:hi], wr[...],
                      preferred_element_type=jnp.float32)
        m = acc[:, :256]
        for t in range(1, 4):
            m = jnp.maximum(m, acc[:, 256 * t:256 * (t + 1)])
        ms.append(m)
    m = jnp.maximum(jnp.maximum(ms[0], ms[1]), jnp.maximum(ms[2], ms[3]))
    ch = jnp.maximum(m[:, :_D_WORD], m[:, _D_WORD:])
    res = wv_ref[...] + jnp.maximum(ch, jnp.float32(0))
    for b in range(_TN // 50):
        out_ref[b] = res[b * 50:(b + 1) * 50, :]


def kernel(X, X_char, word_table, char_table, W3, W5):
    B, S = X.shape
    N = B * S
    n_blk = N // _TN
    flat_words = X.reshape(N).astype(jnp.int32)

    xf = X_char.astype(jnp.float32).reshape(N // 8, 128)
    lane = jnp.arange(128)
    pair_of = jnp.arange(64)
    M = jnp.where(
        (lane[:, None] // 2) == pair_of[None, :],
        jnp.where(lane % 2 == 0, 1.0, 256.0)[:, None], 0.0)
    pids = jnp.dot(xf, M, preferred_element_type=jnp.float32)
    pids = pids.astype(jnp.int32).reshape(N * 8)

    padded = jnp.pad(char_table, ((0, 128), (0, 0)))
    o_part = jnp.repeat(char_table, 64, axis=0)
    parts = []
    for i in range(4):
        parts.append(jnp.tile(padded[i::4], (128, 1)))
        parts.append(o_part)
    pair_table = jnp.concatenate(parts, axis=1)
    pair_table = pair_table.reshape(4 * 8192, 2 * _D_CHAR)

    word_vecs = _sc_gather(N, _D_WORD, 400, True)(flat_words, word_table)
    char_emb = _sc_gather(N * 8, 2 * _D_CHAR, 1600, False)(pids, pair_table)
    ce = char_emb.reshape(N * 2, _D_WORD)

    Wb = jnp.concatenate([_band(W3, 3), _band(W5, 5)], axis=-1)
    Wb = jnp.transpose(Wb, (1, 2, 0, 3)).reshape(256, _L_CHARS * 256)
    Wb = Wb.astype(jnp.bfloat16)
    Wgs = [Wb[lo:hi, 1024 * g:1024 * (g + 1)]
           for g, (lo, hi) in enumerate(_GROUPS)]

    out = pl.pallas_call(
        _conv_body,
        grid=(n_blk,),
        in_specs=[
            pl.BlockSpec((2 * _TN, _D_WORD), lambda i: (i, 0)),
            pl.BlockSpec((_TN, _D_WORD), lambda i: (i, 0)),
        ] + [pl.BlockSpec(w.shape, lambda i: (0, 0)) for w in Wgs],
        out_specs=pl.BlockSpec((_TN // 50, S, _D_WORD), lambda i: (i, 0, 0)),
        out_shape=jax.ShapeDtypeStruct((B, S, _D_WORD), jnp.float32),
    )(ce, word_vecs, *Wgs)

    return out


# speedup vs baseline: 1.6985x; 1.0877x over previous
import functools

import jax
import jax.numpy as jnp
from jax import lax
from jax.experimental import pallas as pl
from jax.experimental.pallas import tpu as pltpu
from jax.experimental.pallas import tpu_sc as plsc

_NC = 2
_NS = 16
_NW = _NC * _NS

_D_CHAR = 16
_L_CHARS = 16
_D_WORD = 128
_TN = 800


def _sc_gather(num_rows, row_dim, chunk, tc_tiling):
    per_w = num_rows // _NW
    n_chunks = per_w // chunk
    mesh = plsc.VectorSubcoreMesh(core_axis_name="c", subcore_axis_name="s")

    @functools.partial(
        pl.kernel,
        out_type=jax.ShapeDtypeStruct((num_rows, row_dim), jnp.float32),
        mesh=mesh,
        compiler_params=pltpu.CompilerParams(use_tc_tiling_on_sc=tc_tiling),
        scratch_types=[
            pltpu.VMEM((chunk,), jnp.int32),
            pltpu.VMEM((chunk,), jnp.int32),
            pltpu.VMEM((chunk, row_dim), jnp.float32),
            pltpu.VMEM((chunk, row_dim), jnp.float32),
            pltpu.SemaphoreType.DMA,
            pltpu.SemaphoreType.DMA,
            pltpu.SemaphoreType.DMA,
            pltpu.SemaphoreType.DMA,
        ],
    )
    def gather(idx_hbm, table_hbm, out_hbm, idx0, idx1, rows0, rows1,
               gsem0, gsem1, osem0, osem1):
        wid = lax.axis_index("s") * _NC + lax.axis_index("c")
        base = wid * per_w
        idx_b, rows_b = [idx0, idx1], [rows0, rows1]
        gsem, osem = [gsem0, gsem1], [osem0, osem1]
        h_g = [None, None]
        h_o = [None, None]
        pltpu.sync_copy(idx_hbm.at[pl.ds(base, chunk)], idx_b[0])
        h_g[0] = pltpu.async_copy(table_hbm.at[idx_b[0]], rows_b[0], gsem[0])
        for c in range(n_chunks):
            cur, nxt = c % 2, (c + 1) % 2
            h_g[cur].wait()
            if c + 1 < n_chunks:
                pltpu.sync_copy(
                    idx_hbm.at[pl.ds(base + (c + 1) * chunk, chunk)],
                    idx_b[nxt])
                if c >= 1:
                    h_o[nxt].wait()
                h_g[nxt] = pltpu.async_copy(
                    table_hbm.at[idx_b[nxt]], rows_b[nxt], gsem[nxt])
            h_o[cur] = pltpu.async_copy(
                rows_b[cur], out_hbm.at[pl.ds(base + c * chunk, chunk)],
                osem[cur])
        h_o[(n_chunks - 1) % 2].wait()
        if n_chunks > 1:
            h_o[n_chunks % 2].wait()

    return gather


def _band(W, k):
    O = W.shape[0]
    T = jnp.transpose(W, (2, 1, 0))
    Tz = jnp.concatenate([T, jnp.zeros((1, _D_CHAR, O), W.dtype)], axis=0)
    t = jnp.arange(_L_CHARS)[:, None]
    p = jnp.arange(_L_CHARS)[None, :]
    dk = p - t + k // 2
    idx = jnp.where((dk >= 0) & (dk < k), dk, k)
    return Tz[idx]


_GROUPS = ((0, 96), (32, 160), (96, 224), (160, 256))


def _conv_body(ce_ref, wv_ref, w0_ref, w1_ref, w2_ref, w3_ref, out_ref):
    x = ce_ref[...].astype(jnp.bfloat16).reshape(_TN, 2 * _D_WORD)
    ms = []
    for wr, (lo, hi) in zip((w0_ref, w1_ref, w2_ref, w3_ref), _GROUPS):
        acc = jnp.dot(x[:, lo:hi], wr[...],
                      preferred_element_type=jnp.float32)
        m = acc[:, :256]
        for t in range(1, 4):
            m = jnp.maximum(m, acc[:, 256 * t:256 * (t + 1)])
        ms.append(m)
    m = jnp.maximum(jnp.maximum(ms[0], ms[1]), jnp.maximum(ms[2], ms[3]))
    ch = jnp.maximum(m[:, :_D_WORD], m[:, _D_WORD:])
    res = wv_ref[...] + jnp.maximum(ch, jnp.float32(0))
    for b in range(_TN // 50):
        out_ref[b] = res[b * 50:(b + 1) * 50, :]


def kernel(X, X_char, word_table, char_table, W3, W5):
    B, S = X.shape
    N = B * S
    n_blk = N // _TN
    flat_words = X.reshape(N).astype(jnp.int32)

    xf = X_char.astype(jnp.float32).reshape(N // 8, 128)
    lane = jnp.arange(128)
    pair_of = jnp.arange(64)
    M = jnp.where(
        (lane[:, None] // 2) == pair_of[None, :],
        jnp.where(lane % 2 == 0, 1.0, 256.0)[:, None], 0.0)
    pids = jnp.dot(xf, M, preferred_element_type=jnp.float32)
    pids = pids.astype(jnp.int32).reshape(N * 8)

    padded = jnp.pad(char_table, ((0, 128), (0, 0)))
    o_part = jnp.repeat(char_table, 64, axis=0)
    parts = []
    for i in range(4):
        parts.append(jnp.tile(padded[i::4], (128, 1)))
        parts.append(o_part)
    pair_table = jnp.concatenate(parts, axis=1)
    pair_table = pair_table.reshape(4 * 8192, 2 * _D_CHAR)

    word_vecs = _sc_gather(N, _D_WORD, 400, True)(flat_words, word_table)
    char_emb = _sc_gather(N * 8, 2 * _D_CHAR, 1600, False)(pids, pair_table)
    ce = char_emb.reshape(N * 2, _D_WORD)

    Wb = jnp.concatenate([_band(W3, 3), _band(W5, 5)], axis=-1)
    Wb = jnp.transpose(Wb, (1, 2, 0, 3)).reshape(256, _L_CHARS * 256)
    Wb = Wb.astype(jnp.bfloat16)
    Wgs = [Wb[lo:hi, 1024 * g:1024 * (g + 1)]
           for g, (lo, hi) in enumerate(_GROUPS)]

    out = pl.pallas_call(
        _conv_body,
        grid=(n_blk,),
        in_specs=[
            pl.BlockSpec((2 * _TN, _D_WORD), lambda i: (i, 0)),
            pl.BlockSpec((_TN, _D_WORD), lambda i: (i, 0)),
        ] + [pl.BlockSpec(w.shape, lambda i: (0, 0)) for w in Wgs],
        out_specs=pl.BlockSpec((_TN // 50, S, _D_WORD), lambda i: (i, 0, 0)),
        out_shape=jax.ShapeDtypeStruct((B, S, _D_WORD), jnp.float32),
    )(ce, word_vecs, *Wgs)

    return out


# speedup vs baseline: 1.7298x; 1.0184x over previous
import functools

import jax
import jax.numpy as jnp
from jax import lax
from jax.experimental import pallas as pl
from jax.experimental.pallas import tpu as pltpu
from jax.experimental.pallas import tpu_sc as plsc

_NC = 2
_NS = 16
_NW = _NC * _NS

_D_CHAR = 16
_L_CHARS = 16
_D_WORD = 128
_TN = 1600


def _sc_gather(num_rows, row_dim, chunk, tc_tiling):
    per_w = num_rows // _NW
    n_chunks = per_w // chunk
    mesh = plsc.VectorSubcoreMesh(core_axis_name="c", subcore_axis_name="s")

    @functools.partial(
        pl.kernel,
        out_type=jax.ShapeDtypeStruct((num_rows, row_dim), jnp.float32),
        mesh=mesh,
        compiler_params=pltpu.CompilerParams(use_tc_tiling_on_sc=tc_tiling),
        scratch_types=[
            pltpu.VMEM((chunk,), jnp.int32),
            pltpu.VMEM((chunk,), jnp.int32),
            pltpu.VMEM((chunk, row_dim), jnp.float32),
            pltpu.VMEM((chunk, row_dim), jnp.float32),
            pltpu.SemaphoreType.DMA,
            pltpu.SemaphoreType.DMA,
            pltpu.SemaphoreType.DMA,
            pltpu.SemaphoreType.DMA,
        ],
    )
    def gather(idx_hbm, table_hbm, out_hbm, idx0, idx1, rows0, rows1,
               gsem0, gsem1, osem0, osem1):
        wid = lax.axis_index("s") * _NC + lax.axis_index("c")
        base = wid * per_w
        idx_b, rows_b = [idx0, idx1], [rows0, rows1]
        gsem, osem = [gsem0, gsem1], [osem0, osem1]
        h_g = [None, None]
        h_o = [None, None]
        pltpu.sync_copy(idx_hbm.at[pl.ds(base, chunk)], idx_b[0])
        h_g[0] = pltpu.async_copy(table_hbm.at[idx_b[0]], rows_b[0], gsem[0])
        for c in range(n_chunks):
            cur, nxt = c % 2, (c + 1) % 2
            h_g[cur].wait()
            if c + 1 < n_chunks:
                pltpu.sync_copy(
                    idx_hbm.at[pl.ds(base + (c + 1) * chunk, chunk)],
                    idx_b[nxt])
                if c >= 1:
                    h_o[nxt].wait()
                h_g[nxt] = pltpu.async_copy(
                    table_hbm.at[idx_b[nxt]], rows_b[nxt], gsem[nxt])
            h_o[cur] = pltpu.async_copy(
                rows_b[cur], out_hbm.at[pl.ds(base + c * chunk, chunk)],
                osem[cur])
        h_o[(n_chunks - 1) % 2].wait()
        if n_chunks > 1:
            h_o[n_chunks % 2].wait()

    return gather


def _band(W, k):
    O = W.shape[0]
    T = jnp.transpose(W, (2, 1, 0))
    Tz = jnp.concatenate([T, jnp.zeros((1, _D_CHAR, O), W.dtype)], axis=0)
    t = jnp.arange(_L_CHARS)[:, None]
    p = jnp.arange(_L_CHARS)[None, :]
    dk = p - t + k // 2
    idx = jnp.where((dk >= 0) & (dk < k), dk, k)
    return Tz[idx]


_GROUPS = ((0, 96), (32, 160), (96, 224), (160, 256))


def _conv_body(ce_ref, wv_ref, w0_ref, w1_ref, w2_ref, w3_ref, out_ref):
    x = ce_ref[...].astype(jnp.bfloat16).reshape(_TN, 2 * _D_WORD)
    ms = []
    for wr, (lo, hi) in zip((w0_ref, w1_ref, w2_ref, w3_ref), _GROUPS):
        acc = jnp.dot(x[:, lo:hi], wr[...],
                      preferred_element_type=jnp.float32)
        m = acc[:, :256]
        for t in range(1, 4):
            m = jnp.maximum(m, acc[:, 256 * t:256 * (t + 1)])
        ms.append(m)
    m = jnp.maximum(jnp.maximum(ms[0], ms[1]), jnp.maximum(ms[2], ms[3]))
    ch = jnp.maximum(m[:, :_D_WORD], m[:, _D_WORD:])
    res = wv_ref[...] + jnp.maximum(ch, jnp.float32(0))
    for b in range(_TN // 50):
        out_ref[b] = res[b * 50:(b + 1) * 50, :]


def kernel(X, X_char, word_table, char_table, W3, W5):
    B, S = X.shape
    N = B * S
    n_blk = N // _TN
    flat_words = X.reshape(N).astype(jnp.int32)

    xf = X_char.astype(jnp.float32).reshape(N // 8, 128)
    lane = jnp.arange(128)
    pair_of = jnp.arange(64)
    M = jnp.where(
        (lane[:, None] // 2) == pair_of[None, :],
        jnp.where(lane % 2 == 0, 1.0, 256.0)[:, None], 0.0)
    pids = jnp.dot(xf, M, preferred_element_type=jnp.float32)
    pids = pids.astype(jnp.int32).reshape(N * 8)

    padded = jnp.pad(char_table, ((0, 128), (0, 0)))
    o_part = jnp.repeat(char_table, 64, axis=0)
    parts = []
    for i in range(4):
        parts.append(jnp.tile(padded[i::4], (128, 1)))
        parts.append(o_part)
    pair_table = jnp.concatenate(parts, axis=1)
    pair_table = pair_table.reshape(4 * 8192, 2 * _D_CHAR)

    word_vecs = _sc_gather(N, _D_WORD, 400, True)(flat_words, word_table)
    char_emb = _sc_gather(N * 8, 2 * _D_CHAR, 1600, False)(pids, pair_table)
    ce = char_emb.reshape(N * 2, _D_WORD)

    Wb = jnp.concatenate([_band(W3, 3), _band(W5, 5)], axis=-1)
    Wb = jnp.transpose(Wb, (1, 2, 0, 3)).reshape(256, _L_CHARS * 256)
    Wb = Wb.astype(jnp.bfloat16)
    Wgs = [Wb[lo:hi, 1024 * g:1024 * (g + 1)]
           for g, (lo, hi) in enumerate(_GROUPS)]

    out = pl.pallas_call(
        _conv_body,
        grid=(n_blk,),
        in_specs=[
            pl.BlockSpec((2 * _TN, _D_WORD), lambda i: (i, 0)),
            pl.BlockSpec((_TN, _D_WORD), lambda i: (i, 0)),
        ] + [pl.BlockSpec(w.shape, lambda i: (0, 0)) for w in Wgs],
        out_specs=pl.BlockSpec((_TN // 50, S, _D_WORD), lambda i: (i, 0, 0)),
        out_shape=jax.ShapeDtypeStruct((B, S, _D_WORD), jnp.float32),
    )(ce, word_vecs, *Wgs)

    return out


# speedup vs baseline: 1.7387x; 1.0051x over previous
import functools

import jax
import jax.numpy as jnp
from jax import lax
from jax.experimental import pallas as pl
from jax.experimental.pallas import tpu as pltpu
from jax.experimental.pallas import tpu_sc as plsc

_NC = 2
_NS = 16
_NW = _NC * _NS

_D_CHAR = 16
_L_CHARS = 16
_D_WORD = 128
_TN = 1600


def _sc_gather(num_rows, row_dim, chunk, tc_tiling):
    per_w = num_rows // _NW
    n_chunks = per_w // chunk
    mesh = plsc.VectorSubcoreMesh(core_axis_name="c", subcore_axis_name="s")

    @functools.partial(
        pl.kernel,
        out_type=jax.ShapeDtypeStruct((num_rows, row_dim), jnp.float32),
        mesh=mesh,
        compiler_params=pltpu.CompilerParams(use_tc_tiling_on_sc=tc_tiling),
        scratch_types=[
            pltpu.VMEM((chunk,), jnp.int32),
            pltpu.VMEM((chunk,), jnp.int32),
            pltpu.VMEM((chunk, row_dim), jnp.float32),
            pltpu.VMEM((chunk, row_dim), jnp.float32),
            pltpu.SemaphoreType.DMA,
            pltpu.SemaphoreType.DMA,
            pltpu.SemaphoreType.DMA,
            pltpu.SemaphoreType.DMA,
        ],
    )
    def gather(idx_hbm, table_hbm, out_hbm, idx0, idx1, rows0, rows1,
               gsem0, gsem1, osem0, osem1):
        wid = lax.axis_index("s") * _NC + lax.axis_index("c")
        base = wid * per_w
        idx_b, rows_b = [idx0, idx1], [rows0, rows1]
        gsem, osem = [gsem0, gsem1], [osem0, osem1]
        h_g = [None, None]
        h_o = [None, None]
        pltpu.sync_copy(idx_hbm.at[pl.ds(base, chunk)], idx_b[0])
        h_g[0] = pltpu.async_copy(table_hbm.at[idx_b[0]], rows_b[0], gsem[0])
        for c in range(n_chunks):
            cur, nxt = c % 2, (c + 1) % 2
            h_g[cur].wait()
            if c + 1 < n_chunks:
                pltpu.sync_copy(
                    idx_hbm.at[pl.ds(base + (c + 1) * chunk, chunk)],
                    idx_b[nxt])
                if c >= 1:
                    h_o[nxt].wait()
                h_g[nxt] = pltpu.async_copy(
                    table_hbm.at[idx_b[nxt]], rows_b[nxt], gsem[nxt])
            h_o[cur] = pltpu.async_copy(
                rows_b[cur], out_hbm.at[pl.ds(base + c * chunk, chunk)],
                osem[cur])
        h_o[(n_chunks - 1) % 2].wait()
        if n_chunks > 1:
            h_o[n_chunks % 2].wait()

    return gather


def _band(W, k):
    O = W.shape[0]
    T = jnp.transpose(W, (2, 1, 0))
    Tz = jnp.concatenate([T, jnp.zeros((1, _D_CHAR, O), W.dtype)], axis=0)
    t = jnp.arange(_L_CHARS)[:, None]
    p = jnp.arange(_L_CHARS)[None, :]
    dk = p - t + k // 2
    idx = jnp.where((dk >= 0) & (dk < k), dk, k)
    return Tz[idx]


_GROUPS = ((0, 96), (32, 160), (96, 224), (160, 256))


def _conv_body(ce_ref, wv_ref, w0_ref, w1_ref, w2_ref, w3_ref, out_ref):
    x = ce_ref[...].astype(jnp.bfloat16).reshape(_TN, 2 * _D_WORD)
    ms = []
    for wr, (lo, hi) in zip((w0_ref, w1_ref, w2_ref, w3_ref), _GROUPS):
        acc = jnp.dot(x[:, lo:hi], wr[...],
                      preferred_element_type=jnp.float32)
        m = acc[:, :256]
        for t in range(1, 4):
            m = jnp.maximum(m, acc[:, 256 * t:256 * (t + 1)])
        ms.append(m)
    m = jnp.maximum(jnp.maximum(ms[0], ms[1]), jnp.maximum(ms[2], ms[3]))
    ch = jnp.maximum(m[:, :_D_WORD], m[:, _D_WORD:])
    res = wv_ref[...] + jnp.maximum(ch, jnp.float32(0))
    for b in range(_TN // 50):
        out_ref[b] = res[b * 50:(b + 1) * 50, :]


def kernel(X, X_char, word_table, char_table, W3, W5):
    B, S = X.shape
    N = B * S
    n_blk = N // _TN
    flat_words = X.reshape(N).astype(jnp.int32)

    xf = X_char.astype(jnp.float32).reshape(N // 16, 256)
    lane = jnp.arange(256)
    pair_of = jnp.arange(128)
    M = jnp.where(
        (lane[:, None] // 2) == pair_of[None, :],
        jnp.where(lane % 2 == 0, 1.0, 256.0)[:, None], 0.0)
    pids = jnp.dot(xf, M, preferred_element_type=jnp.float32)
    pids = pids.astype(jnp.int32).reshape(N * 8)

    padded = jnp.pad(char_table, ((0, 128), (0, 0)))
    o_part = jnp.repeat(char_table, 64, axis=0)
    parts = []
    for i in range(4):
        parts.append(jnp.tile(padded[i::4], (128, 1)))
        parts.append(o_part)
    pair_table = jnp.concatenate(parts, axis=1)
    pair_table = pair_table.reshape(4 * 8192, 2 * _D_CHAR)

    word_vecs = _sc_gather(N, _D_WORD, 400, True)(flat_words, word_table)
    char_emb = _sc_gather(N * 8, 2 * _D_CHAR, 1600, False)(pids, pair_table)
    ce = char_emb.reshape(N * 2, _D_WORD)

    Wb = jnp.concatenate([_band(W3, 3), _band(W5, 5)], axis=-1)
    Wb = jnp.transpose(Wb, (1, 2, 0, 3)).reshape(256, _L_CHARS * 256)
    Wb = Wb.astype(jnp.bfloat16)
    Wgs = [Wb[lo:hi, 1024 * g:1024 * (g + 1)]
           for g, (lo, hi) in enumerate(_GROUPS)]

    out = pl.pallas_call(
        _conv_body,
        grid=(n_blk,),
        in_specs=[
            pl.BlockSpec((2 * _TN, _D_WORD), lambda i: (i, 0)),
            pl.BlockSpec((_TN, _D_WORD), lambda i: (i, 0)),
        ] + [pl.BlockSpec(w.shape, lambda i: (0, 0)) for w in Wgs],
        out_specs=pl.BlockSpec((_TN // 50, S, _D_WORD), lambda i: (i, 0, 0)),
        out_shape=jax.ShapeDtypeStruct((B, S, _D_WORD), jnp.float32),
    )(ce, word_vecs, *Wgs)

    return out
